# Initial kernel scaffold; baseline (speedup 1.0000x reference)
import jax
import jax.numpy as jnp
from jax.experimental import pallas as pl


def kernel(x, segment_ids, W1, b1, W2, b2):
    raise NotImplementedError("write your pallas kernel here")


# speedup vs baseline: 2.5365x; 2.5365x over previous
import functools

import jax
import jax.numpy as jnp
from jax import lax
from jax.experimental import pallas as pl
from jax.experimental.pallas import tpu as pltpu
from jax.experimental.pallas import tpu_sc as plsc

N_ATOMS = 320000
N_IN = 128
N_HIDDEN = 64
N_MOL = 10000


TILE_M = 2560
GRID_M = N_ATOMS // TILE_M


def _mlp_body(x_ref, w1_ref, b1_ref, w2t_ref, b2_ref, y_ref):
    xt = x_ref[...]
    h = jnp.dot(xt, w1_ref[...], preferred_element_type=jnp.float32)
    h = h + b1_ref[...]
    h = h * (1.0 / (1.0 + jnp.exp(-h)))
    yrow = jax.lax.dot_general(
        w2t_ref[...], h, (((1,), (1,)), ((), ())),
        preferred_element_type=jnp.float32)
    yrow = yrow + b2_ref[...]
    y_ref[...] = yrow.reshape(1, 1, TILE_M)


def _mlp(x, W1, b1, W2, b2):
    return pl.pallas_call(
        _mlp_body,
        grid=(GRID_M,),
        in_specs=[
            pl.BlockSpec((TILE_M, N_IN), lambda i: (i, 0)),
            pl.BlockSpec((N_IN, N_HIDDEN), lambda i: (0, 0)),
            pl.BlockSpec((1, N_HIDDEN), lambda i: (0, 0)),
            pl.BlockSpec((1, N_HIDDEN), lambda i: (0, 0)),
            pl.BlockSpec((1, 1), lambda i: (0, 0)),
        ],
        out_specs=pl.BlockSpec((1, 1, TILE_M), lambda i: (i, 0, 0)),
        out_shape=jax.ShapeDtypeStruct((GRID_M, 1, TILE_M), jnp.float32),
    )(x, W1, b1.reshape(1, N_HIDDEN), W2.reshape(1, N_HIDDEN),
      b2.reshape(1, 1))


NS = 16
ATOMS_W = N_ATOMS // NS
CHUNK = 80
NCHUNK = ATOMS_W // CHUNK
ACC = 10240
ACC_W = ACC // NS
FIRE = 10


def _segsum_sc(ids3, y3):
    mesh = plsc.VectorSubcoreMesh(
        core_axis_name="c", subcore_axis_name="s", num_cores=1)

    @functools.partial(
        pl.kernel,
        out_type=jax.ShapeDtypeStruct((ACC,), jnp.float32),
        mesh=mesh,
        scratch_types=[
            pltpu.VMEM((NCHUNK, CHUNK), jnp.int32),
            pltpu.VMEM((NCHUNK, CHUNK), jnp.float32),
            pltpu.VMEM((ACC_W,), jnp.float32),
            pltpu.VMEM_SHARED((ACC,), jnp.float32),
            pltpu.SemaphoreType.DMA,
        ],
    )
    def segsum(ids_hbm, y_hbm, out_hbm, idx_v, y_v, stage_v, acc_sh, sem):
        s = lax.axis_index("s")

        zeros16 = jnp.zeros((16,), jnp.float32)

        def zbody(i, carry):
            stage_v[pl.ds(i * 16, 16)] = zeros16
            return carry

        lax.fori_loop(0, ACC_W // 16, zbody, 0)
        pltpu.sync_copy(stage_v, acc_sh.at[pl.ds(s * ACC_W, ACC_W)])

        pltpu.sync_copy(ids_hbm.at[s], idx_v)
        pltpu.sync_copy(y_hbm.at[s], y_v)

        plsc.subcore_barrier()

        def body(j, carry):
            descs = []
            for b in range(FIRE):
                k = j * FIRE + b
                descs.append(pltpu.async_copy(
                    y_v.at[k], acc_sh.at[idx_v.at[k]], sem, add=True))
            for d in descs:
                d.wait()
            return carry

        lax.fori_loop(0, NCHUNK // FIRE, body, 0)

        plsc.subcore_barrier()

        pltpu.sync_copy(acc_sh.at[pl.ds(s * ACC_W, ACC_W)],
                        out_hbm.at[pl.ds(s * ACC_W, ACC_W)])

    return segsum(ids3, y3)


def kernel(x, segment_ids, W1, b1, W2, b2):
    y = _mlp(x, W1, b1, W2, b2)
    ids3 = segment_ids.astype(jnp.int32).reshape(NS, NCHUNK, CHUNK)
    y3 = y.reshape(NS, NCHUNK, CHUNK)
    agg = _segsum_sc(ids3, y3)
    return agg[:N_MOL]


# speedup vs baseline: 3.9877x; 1.5722x over previous
import functools

import jax
import jax.numpy as jnp
from jax import lax
from jax.experimental import pallas as pl
from jax.experimental.pallas import tpu as pltpu
from jax.experimental.pallas import tpu_sc as plsc

N_ATOMS = 320000
N_IN = 128
N_HIDDEN = 64
N_MOL = 10000


TILE_M = 8192
N_PAD = 327680
GRID_M = N_PAD // TILE_M


def _mlp_body(x_ref, w1_ref, b1_ref, w2t_ref, b2_ref, y_ref):
    i = pl.program_id(0)
    xt = x_ref[...]
    h = jnp.dot(xt, w1_ref[...], preferred_element_type=jnp.float32)
    h = h + b1_ref[...]
    h = h * (1.0 / (1.0 + jnp.exp(-h)))
    yrow = jax.lax.dot_general(
        w2t_ref[...], h, (((1,), (1,)), ((), ())),
        preferred_element_type=jnp.float32)
    yrow = yrow + b2_ref[...]
    g = i * TILE_M + jax.lax.broadcasted_iota(jnp.int32, (1, TILE_M), 1)
    yrow = jnp.where(g < N_ATOMS, yrow, 0.0)
    y_ref[...] = yrow.reshape(TILE_M)


def _mlp(x, W1, b1, W2, b2):
    return pl.pallas_call(
        _mlp_body,
        grid=(GRID_M,),
        in_specs=[
            pl.BlockSpec((TILE_M, N_IN), lambda i: (i, 0)),
            pl.BlockSpec((N_IN, N_HIDDEN), lambda i: (0, 0)),
            pl.BlockSpec((1, N_HIDDEN), lambda i: (0, 0)),
            pl.BlockSpec((1, N_HIDDEN), lambda i: (0, 0)),
            pl.BlockSpec((1, 1), lambda i: (0, 0)),
        ],
        out_specs=pl.BlockSpec((TILE_M,), lambda i: (i,)),
        out_shape=jax.ShapeDtypeStruct((N_PAD,), jnp.float32),
    )(x, W1, b1.reshape(1, N_HIDDEN), W2.reshape(1, N_HIDDEN),
      b2.reshape(1, 1))


NS = 16
ATOMS_W = N_ATOMS // NS
ACC = 10240
ACC_W = ACC // NS


def _segsum_sc(ids, y):
    mesh = plsc.VectorSubcoreMesh(
        core_axis_name="c", subcore_axis_name="s", num_cores=1)

    @functools.partial(
        pl.kernel,
        out_type=jax.ShapeDtypeStruct((ACC,), jnp.float32),
        mesh=mesh,
        scratch_types=[
            pltpu.VMEM((ATOMS_W,), jnp.int32),
            pltpu.VMEM((ATOMS_W,), jnp.float32),
            pltpu.VMEM((ACC_W,), jnp.float32),
            pltpu.VMEM_SHARED((ACC,), jnp.float32),
            pltpu.SemaphoreType.DMA,
        ],
    )
    def segsum(ids_hbm, y_hbm, out_hbm, idx_v, y_v, stage_v, acc_sh, sem):
        s = lax.axis_index("s")

        zeros16 = jnp.zeros((16,), jnp.float32)

        def zbody(i, carry):
            stage_v[pl.ds(i * 16, 16)] = zeros16
            return carry

        lax.fori_loop(0, ACC_W // 16, zbody, 0)
        pltpu.sync_copy(stage_v, acc_sh.at[pl.ds(s * ACC_W, ACC_W)])

        pltpu.sync_copy(ids_hbm.at[pl.ds(s * ATOMS_W, ATOMS_W)], idx_v)
        pltpu.sync_copy(y_hbm.at[pl.ds(s * ATOMS_W, ATOMS_W)], y_v)

        plsc.subcore_barrier()

        pltpu.async_copy(y_v, acc_sh.at[idx_v], sem, add=True).wait()

        plsc.subcore_barrier()

        pltpu.sync_copy(acc_sh.at[pl.ds(s * ACC_W, ACC_W)],
                        out_hbm.at[pl.ds(s * ACC_W, ACC_W)])

    return segsum(ids, y)


def kernel(x, segment_ids, W1, b1, W2, b2):
    y = _mlp(x, W1, b1, W2, b2)
    agg = _segsum_sc(segment_ids.astype(jnp.int32), y)
    return agg[:N_MOL]
